# baseline (speedup 1.0000x reference)
import functools

import jax
import jax.numpy as jnp
from jax import lax
from jax.experimental import pallas as pl
from jax.experimental.pallas import tpu as pltpu
from jax.experimental.pallas import tpu_sc as plsc

B, CIN, H, W = 4, 384, 24, 24
COUT, K = 384, 32
HW = H * W
HWP = 640
L = 16
NC, NS = 2, 16
NW = NC * NS
OPW = COUT // NW
NJ = HW // L
NB1, NB2 = 3, 1


def _make_sc_body(nb):
    nt = OPW * nb

    def _sc_body(xf_hbm, conn_hbm, wb_hbm, bb_hbm, out_hbm,
                 conn_v, w_v, b_v, rows0, rows1, out_v, sem0, sem1):
        wid = lax.axis_index("s") * NC + lax.axis_index("c")

        pltpu.sync_copy(conn_hbm.at[wid], conn_v)
        pltpu.sync_copy(wb_hbm.at[wid], w_v)
        pltpu.sync_copy(bb_hbm.at[wid], b_v)

        def issue(tt, rows_ref, sem):
            oi = tt // nb
            b = lax.rem(tt, nb)
            pltpu.async_copy(xf_hbm.at[b].at[conn_v.at[oi]], rows_ref, sem)

        def wait_rows(rows_ref, sem):
            pltpu.make_async_copy(xf_hbm.at[0].at[pl.ds(0, K)], rows_ref, sem).wait()

        def compute(tt, rows_ref):
            oi = tt // nb
            b = lax.rem(tt, nb)
            bv = b_v[oi]

            ws = [w_v[oi, pl.ds(k * L, L)] for k in range(K)]

            def body(j, _):
                s = pl.ds(oi * HW + j * L, L)
                acc = None
                for g in range(0, K, 8):
                    d = [jnp.abs(rows_ref[g + k, pl.ds(j * L, L)] - ws[g + k]) for k in range(8)]
                    t0 = jnp.maximum(jnp.maximum(d[0], d[1]), jnp.maximum(d[2], d[3]))
                    t1 = jnp.maximum(jnp.maximum(d[4], d[5]), jnp.maximum(d[6], d[7]))
                    t = jnp.maximum(t0, t1)
                    acc = t if acc is None else jnp.maximum(acc, t)
                out_v[b, s] = acc + bv
                return 0

            lax.fori_loop(0, NJ, body, 0)

        issue(0, rows0, sem0)

        def tbody(i, _):
            t0 = i * 2

            @pl.when(t0 + 1 < nt)
            def _():
                issue(t0 + 1, rows1, sem1)

            wait_rows(rows0, sem0)
            compute(t0, rows0)

            @pl.when(t0 + 2 < nt)
            def _():
                issue(t0 + 2, rows0, sem0)

            wait_rows(rows1, sem1)
            compute(t0 + 1, rows1)
            return 0

        lax.fori_loop(0, nt // 2, tbody, 0)

        pltpu.sync_copy(out_v, out_hbm.at[:, wid])

    return _sc_body


def _sc_call(nb):
    mesh = plsc.VectorSubcoreMesh(core_axis_name="c", subcore_axis_name="s")
    return functools.partial(
        pl.kernel,
        out_type=jax.ShapeDtypeStruct((nb, NW, OPW * HW), jnp.float32),
        mesh=mesh,
        scratch_types=[
            pltpu.VMEM((OPW, K), jnp.int32),
            pltpu.VMEM((OPW, K * L), jnp.float32),
            pltpu.VMEM((OPW, L), jnp.float32),
            pltpu.VMEM((K, HWP), jnp.float32),
            pltpu.VMEM((K, HWP), jnp.float32),
            pltpu.VMEM((nb, OPW * HW), jnp.float32),
            pltpu.SemaphoreType.DMA,
            pltpu.SemaphoreType.DMA,
        ],
    )(_make_sc_body(nb))


@jax.jit
def _dist_conv(x, conn3, w_b, bias_b):
    x3 = x.reshape(B, CIN, HW)
    halves = []
    for nb, lo in ((NB1, 0), (NB2, NB1)):
        xf = jnp.pad(lax.slice_in_dim(x3, lo, lo + nb, axis=0),
                     ((0, 0), (0, 0), (0, HWP - HW)))
        out = _sc_call(nb)(xf, conn3, w_b, bias_b)
        halves.append(out.reshape(nb, COUT, H, W))
    return jnp.concatenate(halves, axis=0)


def kernel(x, conn, weights, bias):
    conn3 = conn.reshape(NW, OPW, K)
    w_b = jnp.repeat(weights.reshape(NW, OPW, K), L, axis=-1)
    bias_b = jnp.repeat(bias.reshape(NW, OPW, 1), L, axis=-1)
    return _dist_conv(x, conn3, w_b, bias_b)


# speedup vs baseline: 1.0376x; 1.0376x over previous
import functools

import jax
import jax.numpy as jnp
from jax import lax
from jax.experimental import pallas as pl
from jax.experimental.pallas import tpu as pltpu
from jax.experimental.pallas import tpu_sc as plsc

B, CIN, H, W = 4, 384, 24, 24
COUT, K = 384, 32
HW = H * W
HWP = 640
L = 16
NC, NS = 2, 16
NW = NC * NS
OPW = COUT // NW
NJ = HW // L


def _make_sc_body(nb):
    nt = OPW * nb

    def _sc_body(xf_hbm, conn_hbm, wb_hbm, bb_hbm, out_hbm,
                 conn_v, w_v, b_v, rows0, rows1, out_v, sem0, sem1):
        wid = lax.axis_index("s") * NC + lax.axis_index("c")

        pltpu.sync_copy(conn_hbm.at[wid], conn_v)
        pltpu.sync_copy(wb_hbm.at[wid], w_v)
        pltpu.sync_copy(bb_hbm.at[wid], b_v)

        def issue(tt, rows_ref, sem):
            oi = tt // nb
            b = lax.rem(tt, nb)
            pltpu.async_copy(xf_hbm.at[b].at[conn_v.at[oi]], rows_ref, sem)

        def wait_rows(rows_ref, sem):
            pltpu.make_async_copy(xf_hbm.at[0].at[pl.ds(0, K)], rows_ref, sem).wait()

        def compute(tt, rows_ref):
            oi = tt // nb
            b = lax.rem(tt, nb)
            bv = b_v[oi]

            ws = [w_v[oi, pl.ds(k * L, L)] for k in range(K)]

            def body(j, _):
                s = pl.ds(oi * HW + j * L, L)
                acc = None
                for g in range(0, K, 8):
                    d = [jnp.abs(rows_ref[g + k, pl.ds(j * L, L)] - ws[g + k]) for k in range(8)]
                    t0 = jnp.maximum(jnp.maximum(d[0], d[1]), jnp.maximum(d[2], d[3]))
                    t1 = jnp.maximum(jnp.maximum(d[4], d[5]), jnp.maximum(d[6], d[7]))
                    t = jnp.maximum(t0, t1)
                    acc = t if acc is None else jnp.maximum(acc, t)
                out_v[b, s] = acc + bv
                return 0

            lax.fori_loop(0, NJ, body, 0)

        issue(0, rows0, sem0)

        def tbody(i, _):
            t0 = i * 2

            @pl.when(t0 + 1 < nt)
            def _():
                issue(t0 + 1, rows1, sem1)

            wait_rows(rows0, sem0)
            compute(t0, rows0)

            @pl.when(t0 + 2 < nt)
            def _():
                issue(t0 + 2, rows0, sem0)

            wait_rows(rows1, sem1)
            compute(t0 + 1, rows1)
            return 0

        lax.fori_loop(0, nt // 2, tbody, 0)

        pltpu.sync_copy(out_v, out_hbm.at[:, wid])

    return _sc_body


def _sc_call(nb):
    mesh = plsc.VectorSubcoreMesh(core_axis_name="c", subcore_axis_name="s")
    return functools.partial(
        pl.kernel,
        out_type=jax.ShapeDtypeStruct((nb, NW, OPW * HW), jnp.float32),
        mesh=mesh,
        scratch_types=[
            pltpu.VMEM((OPW, K), jnp.int32),
            pltpu.VMEM((OPW, K * L), jnp.float32),
            pltpu.VMEM((OPW, L), jnp.float32),
            pltpu.VMEM((K, HWP), jnp.float32),
            pltpu.VMEM((K, HWP), jnp.float32),
            pltpu.VMEM((nb, OPW * HW), jnp.float32),
            pltpu.SemaphoreType.DMA,
            pltpu.SemaphoreType.DMA,
        ],
    )(_make_sc_body(nb))


@jax.jit
def _dist_conv(x, conn3, w_b, bias_b):
    x3 = x.reshape(B, CIN, HW)
    xf = jnp.pad(x3, ((0, 0), (0, 0), (0, HWP - HW)))
    out = _sc_call(B)(xf, conn3, w_b, bias_b)
    return out.reshape(B, COUT, H, W)


def kernel(x, conn, weights, bias):
    conn3 = conn.reshape(NW, OPW, K)
    w_b = jnp.repeat(weights.reshape(NW, OPW, K), L, axis=-1)
    bias_b = jnp.repeat(bias.reshape(NW, OPW, 1), L, axis=-1)
    return _dist_conv(x, conn3, w_b, bias_b)


# speedup vs baseline: 1.1398x; 1.0985x over previous
import functools

import jax
import jax.numpy as jnp
from jax import lax
from jax.experimental import pallas as pl
from jax.experimental.pallas import tpu as pltpu
from jax.experimental.pallas import tpu_sc as plsc

B, CIN, H, W = 4, 384, 24, 24
COUT, K = 384, 32
HW = H * W
HWP = 640
L = 16
NC, NS = 2, 16
NW = NC * NS
OPW = COUT // NW
NJ = HW // L


def _make_sc_body(nb):
    nt = OPW * nb

    def _sc_body(xf_hbm, conn_hbm, wb_hbm, bb_hbm, out_hbm,
                 conn_v, w_v, b_v, rows0, rows1, rows2, out_v, sem0, sem1, sem2):
        wid = lax.axis_index("s") * NC + lax.axis_index("c")

        pltpu.sync_copy(conn_hbm.at[wid], conn_v)
        pltpu.sync_copy(wb_hbm.at[wid], w_v)
        pltpu.sync_copy(bb_hbm.at[wid], b_v)

        def issue(tt, rows_ref, sem):
            oi = tt // nb
            b = lax.rem(tt, nb)
            pltpu.async_copy(xf_hbm.at[b].at[conn_v.at[oi]], rows_ref, sem)

        def wait_rows(rows_ref, sem):
            pltpu.make_async_copy(xf_hbm.at[0].at[pl.ds(0, K)], rows_ref, sem).wait()

        def compute(tt, rows_ref):
            oi = tt // nb
            b = lax.rem(tt, nb)
            bv = b_v[oi]

            ws = [w_v[oi, pl.ds(k * L, L)] for k in range(K)]

            def body(j, _):
                s = pl.ds(oi * HW + j * L, L)
                acc = None
                for g in range(0, K, 8):
                    d = [jnp.abs(rows_ref[g + k, pl.ds(j * L, L)] - ws[g + k]) for k in range(8)]
                    t0 = jnp.maximum(jnp.maximum(d[0], d[1]), jnp.maximum(d[2], d[3]))
                    t1 = jnp.maximum(jnp.maximum(d[4], d[5]), jnp.maximum(d[6], d[7]))
                    t = jnp.maximum(t0, t1)
                    acc = t if acc is None else jnp.maximum(acc, t)
                out_v[b, s] = acc + bv
                return 0

            lax.fori_loop(0, NJ, body, 0)

        issue(0, rows0, sem0)
        issue(1, rows1, sem1)

        def tbody(i, _):
            t0 = i * 3
            bufs = ((rows0, sem0), (rows1, sem1), (rows2, sem2))
            for j in range(3):
                rj, sj = bufs[j]
                ra, sa = bufs[(j + 2) % 3]

                @pl.when(t0 + j + 2 < nt)
                def _():
                    issue(t0 + j + 2, ra, sa)

                wait_rows(rj, sj)
                compute(t0 + j, rj)
            return 0

        lax.fori_loop(0, nt // 3, tbody, 0)

        pltpu.sync_copy(out_v, out_hbm.at[:, wid])

    return _sc_body


def _sc_call(nb):
    mesh = plsc.VectorSubcoreMesh(core_axis_name="c", subcore_axis_name="s")
    return functools.partial(
        pl.kernel,
        out_type=jax.ShapeDtypeStruct((nb, NW, OPW * HW), jnp.float32),
        mesh=mesh,
        scratch_types=[
            pltpu.VMEM((OPW, K), jnp.int32),
            pltpu.VMEM((OPW, K * L), jnp.float32),
            pltpu.VMEM((OPW, L), jnp.float32),
            pltpu.VMEM((K, HWP), jnp.float32),
            pltpu.VMEM((K, HWP), jnp.float32),
            pltpu.VMEM((K, HWP), jnp.float32),
            pltpu.VMEM((nb, OPW * HW), jnp.float32),
            pltpu.SemaphoreType.DMA,
            pltpu.SemaphoreType.DMA,
            pltpu.SemaphoreType.DMA,
        ],
    )(_make_sc_body(nb))


@jax.jit
def _dist_conv(x, conn3, w_b, bias_b):
    x3 = x.reshape(B, CIN, HW)
    xf = jnp.pad(x3, ((0, 0), (0, 0), (0, HWP - HW)))
    out = _sc_call(B)(xf, conn3, w_b, bias_b)
    return out.reshape(B, COUT, H, W)


def kernel(x, conn, weights, bias):
    conn3 = conn.reshape(NW, OPW, K)
    w_b = jnp.repeat(weights.reshape(NW, OPW, K), L, axis=-1)
    bias_b = jnp.repeat(bias.reshape(NW, OPW, 1), L, axis=-1)
    return _dist_conv(x, conn3, w_b, bias_b)
